# Initial kernel scaffold; baseline (speedup 1.0000x reference)
import jax
import jax.numpy as jnp
from jax.experimental import pallas as pl


def kernel(x):
    raise NotImplementedError("write your pallas kernel here")


# speedup vs baseline: 23.4463x; 23.4463x over previous
import jax
import jax.numpy as jnp
from jax.experimental import pallas as pl
from jax.experimental.pallas import tpu as pltpu

_FRAME_TOPK = 16
_TOKEN_TOPK = 49
_EMA_ALPHA = 0.9


def _argmax_col(v, iota_col, size):
    m = jnp.max(v, axis=0, keepdims=True)
    cand = jnp.where(v == m, iota_col, size)
    idx = jnp.min(cand, axis=0, keepdims=True)
    oh = (iota_col == idx).astype(jnp.float32)
    return oh, idx


def _frame_repr_kernel(x_ref, out_ref):
    out_ref[...] = jnp.sum(x_ref[...], axis=2) / 196.0


def _frame_sel_kernel(fr_ref, idx_ref, mask_ref):
    T = 64
    fr = fr_ref[0]
    iota_col = jax.lax.broadcasted_iota(jnp.int32, (T, 1), 0)
    iota_row = jax.lax.broadcasted_iota(jnp.int32, (1, T), 1)
    iota_k = jax.lax.broadcasted_iota(jnp.int32, (1, _FRAME_TOPK), 1)

    def ema_body(t, carry):
        ema, nov = carry
        ft = fr_ref[0, pl.ds(t, 1), :]
        d = ft - ema
        n = jnp.sum(d * d)
        nov = jnp.where(iota_col == t, n, nov)
        ema = _EMA_ALPHA * ema + (1.0 - _EMA_ALPHA) * ft
        return ema, nov

    ema0 = fr_ref[0, 0:1, :]
    nov0 = jnp.zeros((T, 1), dtype=jnp.float32)
    _, nov = jax.lax.fori_loop(0, T, ema_body, (ema0, nov0))

    oh, idx = _argmax_col(nov, iota_col, T)
    idx_acc = jnp.where(iota_k == 0, idx, 0)
    mask_row = (iota_row == idx).astype(jnp.float32)
    min_d = jnp.full((T, 1), jnp.inf, dtype=jnp.float32)

    def fps_body(i, carry):
        oh, idx, idx_acc, mask_row, min_d = carry
        last = jnp.sum(oh * fr, axis=0, keepdims=True)
        diff = fr - last
        dist = jnp.sum(diff * diff, axis=1, keepdims=True)
        min_d = jnp.minimum(min_d, dist)
        min_d = jnp.where(oh > 0, -1.0, min_d)
        oh, idx = _argmax_col(min_d, iota_col, T)
        idx_acc = jnp.where(iota_k == i, idx, idx_acc)
        mask_row = mask_row + (iota_row == idx).astype(jnp.float32)
        return oh, idx, idx_acc, mask_row, min_d

    _, _, idx_acc, mask_row, _ = jax.lax.fori_loop(
        1, _FRAME_TOPK, fps_body, (oh, idx, idx_acc, mask_row, min_d))
    idx_ref[0] = idx_acc
    mask_ref[0] = mask_row


def _token_sel_kernel(fidx_ref, x_ref, z_ref, tidx_ref, tmask_ref):
    del fidx_ref
    N = 196
    K = _TOKEN_TOPK
    F = x_ref[0, 0]
    iota_col = jax.lax.broadcasted_iota(jnp.int32, (N, 1), 0)
    iota_row = jax.lax.broadcasted_iota(jnp.int32, (1, N), 1)
    iota_k = jax.lax.broadcasted_iota(jnp.int32, (1, K), 1)
    iota_krow = jax.lax.broadcasted_iota(jnp.int32, (K, 1), 0)

    g = jax.lax.dot_general(F, F, (((1,), (1,)), ((), ())),
                            preferred_element_type=jnp.float32,
                            precision=jax.lax.Precision.HIGHEST)
    ii = jax.lax.broadcasted_iota(jnp.int32, (N, N), 0)
    jj = jax.lax.broadcasted_iota(jnp.int32, (N, N), 1)
    eye = (ii == jj).astype(jnp.float32)
    rn_col = jnp.sum(g * eye, axis=1, keepdims=True)
    rn_row = jnp.sum(g * eye, axis=0, keepdims=True)
    d2 = rn_col + rn_row - 2.0 * g

    gm_col = jnp.mean(g, axis=1, keepdims=True)
    musq = jnp.mean(g)
    d0 = rn_col - 2.0 * gm_col + musq
    oh_col, idx = _argmax_col(d0, iota_col, N)
    oh_row = (iota_row == idx).astype(jnp.float32)

    tidx_acc = jnp.where(iota_k == 0, idx, 0)
    mask_row = oh_row
    sel_mat = (iota_krow == 0).astype(jnp.float32) * oh_row
    min_d = jnp.full((N, 1), jnp.inf, dtype=jnp.float32)

    def fps_body(i, carry):
        oh_col, oh_row, tidx_acc, mask_row, sel_mat, min_d = carry
        dist = jnp.sum(d2 * oh_row, axis=1, keepdims=True)
        min_d = jnp.minimum(min_d, dist)
        min_d = jnp.where(oh_col > 0, -1.0, min_d)
        oh_col, idx = _argmax_col(min_d, iota_col, N)
        oh_row = (iota_row == idx).astype(jnp.float32)
        tidx_acc = jnp.where(iota_k == i, idx, tidx_acc)
        mask_row = mask_row + oh_row
        sel_mat = sel_mat + (iota_krow == i).astype(jnp.float32) * oh_row
        return oh_col, oh_row, tidx_acc, mask_row, sel_mat, min_d

    _, _, tidx_acc, mask_row, sel_mat, _ = jax.lax.fori_loop(
        1, K, fps_body,
        (oh_col, oh_row, tidx_acc, mask_row, sel_mat, min_d))

    z_ref[0, 0] = jax.lax.dot_general(
        sel_mat, F, (((1,), (0,)), ((), ())),
        preferred_element_type=jnp.float32,
                            precision=jax.lax.Precision.HIGHEST)
    tidx_ref[0, 0] = tidx_acc
    tmask_ref[0, 0] = mask_row


def _mask_place_kernel(fidx_ref, cm_ref, out_ref):
    T = 64
    fidx = fidx_ref[0]
    cm = cm_ref[0]
    iota_col = jax.lax.broadcasted_iota(jnp.int32, (T, _FRAME_TOPK), 0)
    p = (iota_col == fidx).astype(jnp.float32)
    out_ref[0] = jax.lax.dot_general(
        p, cm, (((1,), (0,)), ((), ())),
        preferred_element_type=jnp.float32,
                            precision=jax.lax.Precision.HIGHEST)


def kernel(x):
    B, T, N, D = x.shape
    K = _TOKEN_TOPK

    frame_repr = pl.pallas_call(
        _frame_repr_kernel,
        grid=(B, T // 8),
        in_specs=[pl.BlockSpec((1, 8, N, D), lambda b, t: (b, t, 0, 0))],
        out_specs=pl.BlockSpec((1, 8, D), lambda b, t: (b, t, 0)),
        out_shape=jax.ShapeDtypeStruct((B, T, D), jnp.float32),
    )(x)

    frame_idx3, frame_mask3 = pl.pallas_call(
        _frame_sel_kernel,
        grid=(B,),
        in_specs=[pl.BlockSpec((1, T, D), lambda b: (b, 0, 0))],
        out_specs=[
            pl.BlockSpec((1, 1, _FRAME_TOPK), lambda b: (b, 0, 0)),
            pl.BlockSpec((1, 1, T), lambda b: (b, 0, 0)),
        ],
        out_shape=[
            jax.ShapeDtypeStruct((B, 1, _FRAME_TOPK), jnp.int32),
            jax.ShapeDtypeStruct((B, 1, T), jnp.float32),
        ],
    )(frame_repr)
    frame_idx = frame_idx3.reshape(B, _FRAME_TOPK)
    frame_mask = frame_mask3.reshape(B, T)

    grid_spec = pltpu.PrefetchScalarGridSpec(
        num_scalar_prefetch=1,
        grid=(B * _FRAME_TOPK,),
        in_specs=[
            pl.BlockSpec(
                (1, 1, N, D),
                lambda p, idx_ref: (p // _FRAME_TOPK, idx_ref[p], 0, 0)),
        ],
        out_specs=[
            pl.BlockSpec((1, 1, K, D),
                         lambda p, idx_ref: (p // _FRAME_TOPK,
                                             p % _FRAME_TOPK, 0, 0)),
            pl.BlockSpec((1, 1, 1, K),
                         lambda p, idx_ref: (p // _FRAME_TOPK,
                                             p % _FRAME_TOPK, 0, 0)),
            pl.BlockSpec((1, 1, 1, N),
                         lambda p, idx_ref: (p // _FRAME_TOPK,
                                             p % _FRAME_TOPK, 0, 0)),
        ],
    )
    z, tidx4, compact4 = pl.pallas_call(
        _token_sel_kernel,
        grid_spec=grid_spec,
        out_shape=[
            jax.ShapeDtypeStruct((B, _FRAME_TOPK, K, D), jnp.float32),
            jax.ShapeDtypeStruct((B, _FRAME_TOPK, 1, K), jnp.int32),
            jax.ShapeDtypeStruct((B, _FRAME_TOPK, 1, N), jnp.float32),
        ],
    )(frame_idx.reshape(-1), x)
    token_idx = tidx4.reshape(B, _FRAME_TOPK, K)

    token_mask = pl.pallas_call(
        _mask_place_kernel,
        grid=(B,),
        in_specs=[
            pl.BlockSpec((1, 1, _FRAME_TOPK), lambda b: (b, 0, 0)),
            pl.BlockSpec((1, _FRAME_TOPK, N), lambda b: (b, 0, 0)),
        ],
        out_specs=pl.BlockSpec((1, T, N), lambda b: (b, 0, 0)),
        out_shape=jax.ShapeDtypeStruct((B, T, N), jnp.float32),
    )(frame_idx3, compact4.reshape(B, _FRAME_TOPK, N))

    return z, frame_idx, token_idx, frame_mask, token_mask


# speedup vs baseline: 47.2776x; 2.0164x over previous
import jax
import jax.numpy as jnp
from jax import lax
from jax.experimental import pallas as pl
from jax.experimental.pallas import tpu as pltpu
from jax.experimental.pallas import tpu_sc as plsc

_FRAME_TOPK = 16
_TOKEN_TOPK = 49
_EMA_ALPHA = 0.9
_NP = 208
_NEG = -3.0e38


def _argmax_col(v, iota_col, size):
    m = jnp.max(v, axis=0, keepdims=True)
    cand = jnp.where(v == m, iota_col, size)
    idx = jnp.min(cand, axis=0, keepdims=True)
    oh = (iota_col == idx).astype(jnp.float32)
    return oh, idx


def _argmax_row(v, iota_row, size):
    m = jnp.max(v, axis=1, keepdims=True)
    cand = jnp.where(v == m, iota_row, size)
    idx = jnp.min(cand, axis=1, keepdims=True)
    oh = (iota_row == idx).astype(jnp.float32)
    return oh, idx


def _dot(a, b):
    return jax.lax.dot_general(a, b, (((1,), (0,)), ((), ())),
                               preferred_element_type=jnp.float32,
                               precision=jax.lax.Precision.HIGHEST)


def _frame_repr_kernel(x_ref, out_ref):
    out_ref[...] = jnp.sum(x_ref[...], axis=2) / 196.0


def _frame_sel_kernel(fr_ref, idx_ref, mask_ref):
    T = 64
    fr = fr_ref[0]
    iota_col = jax.lax.broadcasted_iota(jnp.int32, (T, 1), 0)
    iota_row = jax.lax.broadcasted_iota(jnp.int32, (1, T), 1)
    iota_k = jax.lax.broadcasted_iota(jnp.int32, (1, _FRAME_TOPK), 1)

    def ema_body(t, carry):
        ema, nov = carry
        ft = fr_ref[0, pl.ds(t, 1), :]
        d = ft - ema
        n = jnp.sum(d * d)
        nov = jnp.where(iota_col == t, n, nov)
        ema = _EMA_ALPHA * ema + (1.0 - _EMA_ALPHA) * ft
        return ema, nov

    ema0 = fr_ref[0, 0:1, :]
    nov0 = jnp.zeros((T, 1), dtype=jnp.float32)
    _, nov = jax.lax.fori_loop(0, T, ema_body, (ema0, nov0))

    oh, idx = _argmax_col(nov, iota_col, T)
    idx_acc = jnp.where(iota_k == 0, idx, 0)
    mask_row = (iota_row == idx).astype(jnp.float32)
    min_d = jnp.full((T, 1), jnp.inf, dtype=jnp.float32)

    def fps_body(i, carry):
        oh, idx, idx_acc, mask_row, min_d = carry
        last = jnp.sum(oh * fr, axis=0, keepdims=True)
        diff = fr - last
        dist = jnp.sum(diff * diff, axis=1, keepdims=True)
        min_d = jnp.minimum(min_d, dist)
        min_d = jnp.where(oh > 0, -1.0, min_d)
        oh, idx = _argmax_col(min_d, iota_col, T)
        idx_acc = jnp.where(iota_k == i, idx, idx_acc)
        mask_row = mask_row + (iota_row == idx).astype(jnp.float32)
        return oh, idx, idx_acc, mask_row, min_d

    _, _, idx_acc, mask_row, _ = jax.lax.fori_loop(
        1, _FRAME_TOPK, fps_body, (oh, idx, idx_acc, mask_row, min_d))
    idx_ref[0] = idx_acc
    mask_ref[0] = mask_row


def _d2_kernel(fidx_ref, x_ref, d2_ref, d0_ref):
    del fidx_ref
    F = x_ref[0, 0]
    Fp = jnp.concatenate(
        [F, jnp.zeros((_NP - 196, F.shape[1]), jnp.float32)], axis=0)
    g = jax.lax.dot_general(Fp, Fp, (((1,), (1,)), ((), ())),
                            preferred_element_type=jnp.float32,
                            precision=jax.lax.Precision.HIGHEST)
    ii = jax.lax.broadcasted_iota(jnp.int32, (_NP, _NP), 0)
    jj = jax.lax.broadcasted_iota(jnp.int32, (_NP, _NP), 1)
    eye = (ii == jj).astype(jnp.float32)
    rn_col = jnp.sum(g * eye, axis=1, keepdims=True)
    rn_row = jnp.sum(g * eye, axis=0, keepdims=True)
    d2 = rn_col + rn_row - 2.0 * g
    pad = jnp.logical_or(ii >= 196, jj >= 196)
    d2_ref[0] = jnp.where(pad, _NEG, d2)

    gm_row = jnp.sum(g, axis=0, keepdims=True) * (1.0 / 196.0)
    musq = jnp.sum(g) * (1.0 / (196.0 * 196.0))
    d0 = rn_row - 2.0 * gm_row + musq
    jr = jax.lax.broadcasted_iota(jnp.int32, (1, _NP), 1)
    d0_ref[0] = jnp.where(jr >= 196, _NEG, d0)


def _fps_tok_kernel(d2_ref, d0_ref, oh_ref, tidx_ref):
    K = _TOKEN_TOPK
    d2 = d2_ref[0]
    d0 = d0_ref[0]
    iota_col = jax.lax.broadcasted_iota(jnp.int32, (_NP, 1), 0)
    iota_row = jax.lax.broadcasted_iota(jnp.int32, (1, _NP), 1)
    iota_k = jax.lax.broadcasted_iota(jnp.int32, (1, 64), 1)
    iota_krow = jax.lax.broadcasted_iota(jnp.int32, (64, 1), 0)

    oh_row, idx = _argmax_row(d0, iota_row, _NP)
    oh_col = (iota_col == idx).astype(jnp.float32)
    tidx_acc = jnp.where(iota_k == 0, idx, 0)
    oh_acc = (iota_krow == 0).astype(jnp.float32) * oh_row
    min_d = jnp.full((_NP, 1), jnp.inf, dtype=jnp.float32)

    def fps_body(i, carry):
        oh_col, oh_row, tidx_acc, oh_acc, min_d = carry
        dist = jnp.sum(d2 * oh_row, axis=1, keepdims=True)
        min_d = jnp.minimum(min_d, dist)
        min_d = jnp.where(oh_col > 0, -1.0, min_d)
        oh_col, idx = _argmax_col(min_d, iota_col, _NP)
        oh_row = (iota_row == idx).astype(jnp.float32)
        tidx_acc = jnp.where(iota_k == i, idx, tidx_acc)
        oh_acc = oh_acc + (iota_krow == i).astype(jnp.float32) * oh_row
        return oh_col, oh_row, tidx_acc, oh_acc, min_d

    _, _, tidx_acc, oh_acc, _ = jax.lax.fori_loop(
        1, K, fps_body, (oh_col, oh_row, tidx_acc, oh_acc, min_d))
    oh_ref[0] = oh_acc
    tidx_ref[0] = tidx_acc


def _zc_kernel(fidx_ref, oh_ref, x_ref, z_ref, cm_ref):
    del fidx_ref
    F = x_ref[0, 0]
    ohm = oh_ref[0][:, :196]
    z64 = _dot(ohm, F)
    z_ref[0, 0] = z64[:_TOKEN_TOPK]
    cm_ref[0, 0] = jnp.sum(ohm, axis=0, keepdims=True)


def _mask_place_kernel(fidx_ref, cm_ref, out_ref):
    T = 64
    fidx = fidx_ref[0]
    cm = cm_ref[0]
    iota_col = jax.lax.broadcasted_iota(jnp.int32, (T, _FRAME_TOPK), 0)
    p = (iota_col == fidx).astype(jnp.float32)
    out_ref[0] = _dot(p, cm)


def _fps_tok_sc_body(d2_hbm, d0_hbm, oh_hbm, tidx_hbm,
                     d2_v, d0_v, min_v, oh_v, tidx_v):
    K = _TOKEN_TOPK
    C = _NP // 16
    p = lax.axis_index("s") * 2 + lax.axis_index("c")
    pltpu.sync_copy(d2_hbm.at[p], d2_v)
    pltpu.sync_copy(d0_hbm.at[p], d0_v)
    iota = lax.iota(jnp.int32, 16)

    def write_sel(k, sel):
        for c in range(C):
            oh_v[k, pl.ds(c * 16, 16)] = jnp.where(
                iota + (c * 16) == sel, 1.0, 0.0).astype(jnp.float32)
        for c in range(4):
            cur = tidx_v[pl.ds(c * 16, 16)]
            tidx_v[pl.ds(c * 16, 16)] = jnp.where(
                iota + (c * 16) == k, sel, cur)

    for c in range(4):
        tidx_v[pl.ds(c * 16, 16)] = jnp.zeros((16,), jnp.int32)
    for c in range(C):
        min_v[pl.ds(c * 16, 16)] = jnp.full((16,), jnp.inf, jnp.float32)

    def argmax_chunks(vecs):
        vmax = vecs[0]
        for v in vecs[1:]:
            vmax = jnp.maximum(vmax, v)
        gmax = vmax[0]
        for l in range(1, 16):
            gmax = jnp.maximum(gmax, vmax[l])
        cmin = jnp.where(vecs[0] == gmax, iota, _NP)
        for c in range(1, len(vecs)):
            cand = jnp.where(vecs[c] == gmax, iota + (c * 16), _NP)
            cmin = jnp.minimum(cmin, cand)
        sel = cmin[0]
        for l in range(1, 16):
            sel = jnp.minimum(sel, cmin[l])
        return sel

    sel0 = argmax_chunks([d0_v[pl.ds(c * 16, 16)] for c in range(C)])
    write_sel(0, sel0)

    def body(k, sel):
        ms = []
        for c in range(C):
            row = d2_v[sel, pl.ds(c * 16, 16)]
            m = jnp.minimum(min_v[pl.ds(c * 16, 16)], row)
            m = jnp.where(iota + (c * 16) == sel, -1.0, m)
            min_v[pl.ds(c * 16, 16)] = m
            ms.append(m)
        new = argmax_chunks(ms)
        write_sel(k, new)
        return new

    lax.fori_loop(1, K, body, sel0)

    def zero_row(k, carry):
        for c in range(C):
            oh_v[k, pl.ds(c * 16, 16)] = jnp.zeros((16,), jnp.float32)
        return carry

    lax.fori_loop(K, 64, zero_row, 0)
    pltpu.sync_copy(oh_v, oh_hbm.at[p])
    pltpu.sync_copy(tidx_v, tidx_hbm.at[p])


def _run_fps_tok_sc(d2, d0):
    P = d2.shape[0]
    mesh = plsc.VectorSubcoreMesh(core_axis_name="c", subcore_axis_name="s")
    fps = pl.kernel(
        _fps_tok_sc_body,
        out_type=[
            jax.ShapeDtypeStruct((P, 64, _NP), jnp.float32),
            jax.ShapeDtypeStruct((P, 64), jnp.int32),
        ],
        mesh=mesh,
        scratch_types=[
            pltpu.VMEM((_NP, _NP), jnp.float32),
            pltpu.VMEM((_NP,), jnp.float32),
            pltpu.VMEM((_NP,), jnp.float32),
            pltpu.VMEM((64, _NP), jnp.float32),
            pltpu.VMEM((64,), jnp.int32),
        ],
    )
    oh, tidx = fps(d2, d0.reshape(P, _NP))
    return oh, tidx.reshape(P, 1, 64)


def _run_fps_tok(d2, d0):
    P = d2.shape[0]
    return pl.pallas_call(
        _fps_tok_kernel,
        grid=(P,),
        in_specs=[
            pl.BlockSpec((1, _NP, _NP), lambda p: (p, 0, 0)),
            pl.BlockSpec((1, 1, _NP), lambda p: (p, 0, 0)),
        ],
        out_specs=[
            pl.BlockSpec((1, 64, _NP), lambda p: (p, 0, 0)),
            pl.BlockSpec((1, 1, 64), lambda p: (p, 0, 0)),
        ],
        out_shape=[
            jax.ShapeDtypeStruct((P, 64, _NP), jnp.float32),
            jax.ShapeDtypeStruct((P, 1, 64), jnp.int32),
        ],
    )(d2, d0)


def kernel(x):
    B, T, N, D = x.shape
    K = _TOKEN_TOPK
    P = B * _FRAME_TOPK

    frame_repr = pl.pallas_call(
        _frame_repr_kernel,
        grid=(B, T // 8),
        in_specs=[pl.BlockSpec((1, 8, N, D), lambda b, t: (b, t, 0, 0))],
        out_specs=pl.BlockSpec((1, 8, D), lambda b, t: (b, t, 0)),
        out_shape=jax.ShapeDtypeStruct((B, T, D), jnp.float32),
    )(x)

    frame_idx3, frame_mask3 = pl.pallas_call(
        _frame_sel_kernel,
        grid=(B,),
        in_specs=[pl.BlockSpec((1, T, D), lambda b: (b, 0, 0))],
        out_specs=[
            pl.BlockSpec((1, 1, _FRAME_TOPK), lambda b: (b, 0, 0)),
            pl.BlockSpec((1, 1, T), lambda b: (b, 0, 0)),
        ],
        out_shape=[
            jax.ShapeDtypeStruct((B, 1, _FRAME_TOPK), jnp.int32),
            jax.ShapeDtypeStruct((B, 1, T), jnp.float32),
        ],
    )(frame_repr)
    frame_idx = frame_idx3.reshape(B, _FRAME_TOPK)
    frame_mask = frame_mask3.reshape(B, T)
    fidx_flat = frame_idx.reshape(-1)

    d2, d0 = pl.pallas_call(
        _d2_kernel,
        grid_spec=pltpu.PrefetchScalarGridSpec(
            num_scalar_prefetch=1,
            grid=(P,),
            in_specs=[
                pl.BlockSpec(
                    (1, 1, N, D),
                    lambda p, idx_ref: (p // _FRAME_TOPK, idx_ref[p], 0, 0)),
            ],
            out_specs=[
                pl.BlockSpec((1, _NP, _NP), lambda p, idx_ref: (p, 0, 0)),
                pl.BlockSpec((1, 1, _NP), lambda p, idx_ref: (p, 0, 0)),
            ],
        ),
        out_shape=[
            jax.ShapeDtypeStruct((P, _NP, _NP), jnp.float32),
            jax.ShapeDtypeStruct((P, 1, _NP), jnp.float32),
        ],
    )(fidx_flat, x)

    oh, tidx = _run_fps_tok_sc(d2, d0)
    token_idx = tidx.reshape(B, _FRAME_TOPK, 64)[:, :, :K]

    z, compact4 = pl.pallas_call(
        _zc_kernel,
        grid_spec=pltpu.PrefetchScalarGridSpec(
            num_scalar_prefetch=1,
            grid=(P,),
            in_specs=[
                pl.BlockSpec((1, 64, _NP), lambda p, idx_ref: (p, 0, 0)),
                pl.BlockSpec(
                    (1, 1, N, D),
                    lambda p, idx_ref: (p // _FRAME_TOPK, idx_ref[p], 0, 0)),
            ],
            out_specs=[
                pl.BlockSpec((1, 1, K, D),
                             lambda p, idx_ref: (p // _FRAME_TOPK,
                                                 p % _FRAME_TOPK, 0, 0)),
                pl.BlockSpec((1, 1, 1, N),
                             lambda p, idx_ref: (p // _FRAME_TOPK,
                                                 p % _FRAME_TOPK, 0, 0)),
            ],
        ),
        out_shape=[
            jax.ShapeDtypeStruct((B, _FRAME_TOPK, K, D), jnp.float32),
            jax.ShapeDtypeStruct((B, _FRAME_TOPK, 1, N), jnp.float32),
        ],
    )(fidx_flat, oh, x)

    token_mask = pl.pallas_call(
        _mask_place_kernel,
        grid=(B,),
        in_specs=[
            pl.BlockSpec((1, 1, _FRAME_TOPK), lambda b: (b, 0, 0)),
            pl.BlockSpec((1, _FRAME_TOPK, N), lambda b: (b, 0, 0)),
        ],
        out_specs=pl.BlockSpec((1, T, N), lambda b: (b, 0, 0)),
        out_shape=jax.ShapeDtypeStruct((B, T, N), jnp.float32),
    )(frame_idx3, compact4.reshape(B, _FRAME_TOPK, N))

    return z, frame_idx, token_idx, frame_mask, token_mask
